# Initial kernel scaffold; baseline (speedup 1.0000x reference)
import jax
import jax.numpy as jnp
from jax.experimental import pallas as pl


def kernel(source_feature, source_label, target_feature, target_prediction, target_reliability, source_prototypes, target_prototypes):
    raise NotImplementedError("write your pallas kernel here")


# speedup vs baseline: 6.3599x; 6.3599x over previous
import functools

import jax
import jax.numpy as jnp
from jax.experimental import pallas as pl
from jax.experimental.pallas import tpu as pltpu

TYPE_NUM = 100
KPAD = 128
FEATURE_DIM = 1024
MOMENTUM = 0.9
N = 16384
BLOCK = 2048
NB = N // BLOCK


def _body(src_ref, lab_ref, tgt_ref, pred_ref, rel_ref, psrc_ref, ptgt_ref,
          out_ref, acc_src, acc_tgt, cnt_src, cnt_tgt):
    i = pl.program_id(0)

    @pl.when(i == 0)
    def _init():
        acc_src[...] = jnp.zeros_like(acc_src)
        acc_tgt[...] = jnp.zeros_like(acc_tgt)
        cnt_src[...] = jnp.zeros_like(cnt_src)
        cnt_tgt[...] = jnp.zeros_like(cnt_tgt)

    classes = jax.lax.broadcasted_iota(jnp.int32, (BLOCK, KPAD), 1)
    lab = lab_ref[0, 0, :]
    pred = pred_ref[0, 0, :]
    rel = rel_ref[0, 0, :]

    oh_src = (lab[:, None] == classes).astype(jnp.float32)
    oh_tgt = (pred[:, None] == classes).astype(jnp.float32)

    dn = (((0,), (0,)), ((), ()))
    acc_src[...] += jax.lax.dot_general(
        oh_src, src_ref[...], dn, preferred_element_type=jnp.float32)
    acc_tgt[...] += jax.lax.dot_general(
        oh_tgt * rel[:, None], tgt_ref[...], dn,
        preferred_element_type=jnp.float32)
    cnt_src[...] += jnp.sum(oh_src, axis=0, keepdims=True)
    cnt_tgt[...] += jnp.sum(oh_tgt, axis=0, keepdims=True)

    @pl.when(i == NB - 1)
    def _epilogue():
        csrc = cnt_src[...].reshape(KPAD, 1)
        ctgt = cnt_tgt[...].reshape(KPAD, 1)
        psrc = psrc_ref[...]
        ptgt = ptgt_ref[...]

        src_mean = acc_src[...] / jnp.maximum(csrc, 1.0)
        new_src = jnp.where(csrc > 0.0,
                            MOMENTUM * psrc + (1.0 - MOMENTUM) * src_mean,
                            psrc)

        tgt_mean = acc_tgt[...] / jnp.maximum(ctgt, 1.0)
        proto_nonzero = (jnp.sum(jnp.abs(ptgt), axis=1, keepdims=True) > 1e-07)
        updated = jnp.where(proto_nonzero,
                            MOMENTUM * ptgt + (1.0 - MOMENTUM) * tgt_mean,
                            tgt_mean)
        new_tgt = jnp.where(ctgt > 0.0, updated, ptgt)

        ns = new_src / jnp.maximum(
            jnp.sqrt(jnp.sum(new_src * new_src, axis=1, keepdims=True)), 1e-12)
        nt = new_tgt / jnp.maximum(
            jnp.sqrt(jnp.sum(new_tgt * new_tgt, axis=1, keepdims=True)), 1e-12)
        diff = ns - nt
        loss = jnp.sum(diff * diff) / float(TYPE_NUM * FEATURE_DIM)
        present = jnp.sum(
            (jnp.sum(jnp.abs(new_tgt), axis=1) > 1e-07).astype(jnp.float32))
        loss = loss * (present >= float(TYPE_NUM)).astype(jnp.float32)
        out_ref[...] = loss.reshape(1, 1)


@jax.jit
def _align_loss(source_feature, lab3, target_feature, pred3, rel3,
                psrc_pad, ptgt_pad):
    out = pl.pallas_call(
        _body,
        grid=(NB,),
        in_specs=[
            pl.BlockSpec((BLOCK, FEATURE_DIM), lambda i: (i, 0)),
            pl.BlockSpec((1, 1, BLOCK), lambda i: (i, 0, 0)),
            pl.BlockSpec((BLOCK, FEATURE_DIM), lambda i: (i, 0)),
            pl.BlockSpec((1, 1, BLOCK), lambda i: (i, 0, 0)),
            pl.BlockSpec((1, 1, BLOCK), lambda i: (i, 0, 0)),
            pl.BlockSpec((KPAD, FEATURE_DIM), lambda i: (0, 0)),
            pl.BlockSpec((KPAD, FEATURE_DIM), lambda i: (0, 0)),
        ],
        out_specs=pl.BlockSpec((1, 1), lambda i: (0, 0)),
        out_shape=jax.ShapeDtypeStruct((1, 1), jnp.float32),
        scratch_shapes=[
            pltpu.VMEM((KPAD, FEATURE_DIM), jnp.float32),
            pltpu.VMEM((KPAD, FEATURE_DIM), jnp.float32),
            pltpu.VMEM((1, KPAD), jnp.float32),
            pltpu.VMEM((1, KPAD), jnp.float32),
        ],
    )(source_feature, lab3, target_feature, pred3, rel3, psrc_pad, ptgt_pad)
    return out[0, 0]


def kernel(source_feature, source_label, target_feature, target_prediction,
           target_reliability, source_prototypes, target_prototypes):
    lab3 = source_label.astype(jnp.int32).reshape(NB, 1, BLOCK)
    pred3 = target_prediction.astype(jnp.int32).reshape(NB, 1, BLOCK)
    rel3 = target_reliability.reshape(NB, 1, BLOCK)
    pad = ((0, KPAD - TYPE_NUM), (0, 0))
    psrc_pad = jnp.pad(source_prototypes, pad)
    ptgt_pad = jnp.pad(target_prototypes, pad)
    return _align_loss(source_feature, lab3, target_feature, pred3, rel3,
                       psrc_pad, ptgt_pad)
